# baseline (speedup 1.0000x reference)
import jax
import jax.numpy as jnp
from jax.experimental import pallas as pl
from jax.experimental.pallas import tpu as pltpu

_HIDDEN = 64
_N = 128
_ITERS = 2
_BB = 8


def _dot(a, b):
    return jax.lax.dot_general(a.astype(jnp.bfloat16), b.astype(jnp.bfloat16),
                               (((1,), (0,)), ((), ())),
                               preferred_element_type=jnp.float32)


def _mpnn_kernel(x_ref, Wemb_ref, bemb_ref, Wadj_ref, Wmsg_ref, bmsg_ref,
                 Wz_ref, Uz_ref, bz_ref, Wr_ref, Ur_ref, br_ref,
                 Wh_ref, Uh_ref, bh_ref, Wro_ref, bro_ref,
                 out_ref, A_ref):
    x = x_ref[...]
    h = jnp.tanh(_dot(x, Wemb_ref[...]) + bemb_ref[...])
    inv_sqrt = 1.0 / (float(_HIDDEN) ** 0.5)
    for t in range(_ITERS):
        hw = _dot(h, Wadj_ref[t]) * inv_sqrt
        pre = _dot(h, Wmsg_ref[t]) + bmsg_ref[t]
        msgs = []
        for b in range(_BB):
            sl = slice(b * _N, (b + 1) * _N)
            hb = h[sl, :]
            logits = jax.lax.dot_general(
                hw[sl, :].astype(jnp.bfloat16), hb.astype(jnp.bfloat16),
                (((1,), (1,)), ((), ())),
                preferred_element_type=jnp.float32)
            m = jnp.max(logits, axis=-1, keepdims=True)
            e = jnp.exp(logits - m)
            Ab = e / jnp.sum(e, axis=-1, keepdims=True)
            if t == _ITERS - 1:
                A_ref[b] = Ab
            msgs.append(_dot(Ab, pre[sl, :]))
        msg = jnp.tanh(jnp.concatenate(msgs, axis=0))
        z = jax.nn.sigmoid(_dot(msg, Wz_ref[t]) + _dot(h, Uz_ref[t]) + bz_ref[t])
        r = jax.nn.sigmoid(_dot(msg, Wr_ref[t]) + _dot(h, Ur_ref[t]) + br_ref[t])
        htil = jnp.tanh(_dot(msg, Wh_ref[t]) + _dot(r * h, Uh_ref[t]) + bh_ref[t])
        h = (1.0 - z) * h + z * htil
    pooled = jnp.concatenate(
        [jnp.sum(h[b * _N:(b + 1) * _N, :], axis=0, keepdims=True)
         for b in range(_BB)], axis=0)
    out_ref[...] = jnp.tanh(_dot(pooled, Wro_ref[...]) + bro_ref[...])


def kernel(jets, W_emb, b_emb, W_adj, W_msg, b_msg,
           Wz, Uz, bz, Wr, Ur, br, Wh, Uh, bh, W_ro, b_ro):
    B, N, F = jets.shape
    H = _HIDDEN
    x = jnp.concatenate([jets, jnp.ones((B, N, 1), jets.dtype)], axis=-1)
    x = x.reshape(B * N, F + 1)

    def rep(ix):
        return pl.BlockSpec(ix, lambda i: (0,) * len(ix))

    out, A = pl.pallas_call(
        _mpnn_kernel,
        grid=(B // _BB,),
        in_specs=[
            pl.BlockSpec((_BB * N, F + 1), lambda i: (i, 0)),
            rep((F + 1, H)),
            rep((1, H)),
            rep((_ITERS, H, H)),
            rep((_ITERS, H, H)),
            rep((_ITERS, 1, H)),
            rep((_ITERS, H, H)), rep((_ITERS, H, H)), rep((_ITERS, 1, H)),
            rep((_ITERS, H, H)), rep((_ITERS, H, H)), rep((_ITERS, 1, H)),
            rep((_ITERS, H, H)), rep((_ITERS, H, H)), rep((_ITERS, 1, H)),
            rep((H, H)),
            rep((1, H)),
        ],
        out_specs=[
            pl.BlockSpec((_BB, H), lambda i: (i, 0)),
            pl.BlockSpec((_BB, N, N), lambda i: (i, 0, 0)),
        ],
        out_shape=[
            jax.ShapeDtypeStruct((B, H), jnp.float32),
            jax.ShapeDtypeStruct((B, N, N), jnp.float32),
        ],
        compiler_params=pltpu.CompilerParams(
            dimension_semantics=("parallel",)),
    )(x, W_emb, b_emb.reshape(1, H),
      W_adj, W_msg, b_msg.reshape(_ITERS, 1, H),
      Wz, Uz, bz.reshape(_ITERS, 1, H),
      Wr, Ur, br.reshape(_ITERS, 1, H),
      Wh, Uh, bh.reshape(_ITERS, 1, H),
      W_ro, b_ro.reshape(1, H))
    return (out, A)


# speedup vs baseline: 1.5116x; 1.5116x over previous
import jax
import jax.numpy as jnp
from jax.experimental import pallas as pl
from jax.experimental.pallas import tpu as pltpu

_HIDDEN = 64
_N = 128
_ITERS = 2
_BB = 8


def _dot(a, b):
    return jax.lax.dot_general(a, b, (((1,), (0,)), ((), ())),
                               preferred_element_type=jnp.float32)


def _mpnn_kernel(x_ref, Wemb_ref, bemb_ref, Wadj_ref, Wmsg_ref, bmsg_ref,
                 Wz_ref, Uz_ref, bz_ref, Wr_ref, Ur_ref, br_ref,
                 Wh_ref, Uh_ref, bh_ref, Wro_ref, bro_ref,
                 out_ref, A_ref):
    x = x_ref[...]
    h = jnp.tanh(_dot(x, Wemb_ref[...]) + bemb_ref[...])
    inv_sqrt = 1.0 / (float(_HIDDEN) ** 0.5)
    ones_blk = jnp.ones((_N, _HIDDEN), jnp.float32)
    for t in range(_ITERS):
        hw = _dot(h, Wadj_ref[t]) * inv_sqrt
        pre = _dot(h, Wmsg_ref[t]) + bmsg_ref[t]
        msgs = []
        for b in range(_BB):
            sl = slice(b * _N, (b + 1) * _N)
            logits = jax.lax.dot_general(
                hw[sl, :], h[sl, :], (((1,), (1,)), ((), ())),
                preferred_element_type=jnp.float32)
            e = jnp.exp(logits)
            pre_aug = jnp.concatenate([pre[sl, :], ones_blk], axis=1)
            s = _dot(e, pre_aug)
            inv = 1.0 / s[:, _HIDDEN:]
            msgs.append(s[:, :_HIDDEN] * inv)
            if t == _ITERS - 1:
                A_ref[b] = e * jnp.concatenate([inv, inv], axis=1)
        msg = jnp.tanh(jnp.concatenate(msgs, axis=0))
        z = jax.nn.sigmoid(_dot(msg, Wz_ref[t]) + _dot(h, Uz_ref[t]) + bz_ref[t])
        r = jax.nn.sigmoid(_dot(msg, Wr_ref[t]) + _dot(h, Ur_ref[t]) + br_ref[t])
        htil = jnp.tanh(_dot(msg, Wh_ref[t]) + _dot(r * h, Uh_ref[t]) + bh_ref[t])
        h = (1.0 - z) * h + z * htil
    pooled = jnp.concatenate(
        [jnp.sum(h[b * _N:(b + 1) * _N, :], axis=0, keepdims=True)
         for b in range(_BB)], axis=0)
    out_ref[...] = jnp.tanh(_dot(pooled, Wro_ref[...]) + bro_ref[...])


def kernel(jets, W_emb, b_emb, W_adj, W_msg, b_msg,
           Wz, Uz, bz, Wr, Ur, br, Wh, Uh, bh, W_ro, b_ro):
    B, N, F = jets.shape
    H = _HIDDEN
    x = jnp.concatenate([jets, jnp.ones((B, N, 1), jets.dtype)], axis=-1)
    x = x.reshape(B * N, F + 1)

    def rep(ix):
        return pl.BlockSpec(ix, lambda i: (0,) * len(ix))

    out, A = pl.pallas_call(
        _mpnn_kernel,
        grid=(B // _BB,),
        in_specs=[
            pl.BlockSpec((_BB * N, F + 1), lambda i: (i, 0)),
            rep((F + 1, H)),
            rep((1, H)),
            rep((_ITERS, H, H)),
            rep((_ITERS, H, H)),
            rep((_ITERS, 1, H)),
            rep((_ITERS, H, H)), rep((_ITERS, H, H)), rep((_ITERS, 1, H)),
            rep((_ITERS, H, H)), rep((_ITERS, H, H)), rep((_ITERS, 1, H)),
            rep((_ITERS, H, H)), rep((_ITERS, H, H)), rep((_ITERS, 1, H)),
            rep((H, H)),
            rep((1, H)),
        ],
        out_specs=[
            pl.BlockSpec((_BB, H), lambda i: (i, 0)),
            pl.BlockSpec((_BB, N, N), lambda i: (i, 0, 0)),
        ],
        out_shape=[
            jax.ShapeDtypeStruct((B, H), jnp.float32),
            jax.ShapeDtypeStruct((B, N, N), jnp.float32),
        ],
        compiler_params=pltpu.CompilerParams(
            dimension_semantics=("parallel",)),
    )(x, W_emb, b_emb.reshape(1, H),
      W_adj, W_msg, b_msg.reshape(_ITERS, 1, H),
      Wz, Uz, bz.reshape(_ITERS, 1, H),
      Wr, Ur, br.reshape(_ITERS, 1, H),
      Wh, Uh, bh.reshape(_ITERS, 1, H),
      W_ro, b_ro.reshape(1, H))
    return (out, A)
